# baseline (speedup 1.0000x reference)
import functools

import jax
import jax.numpy as jnp
from jax import lax
from jax.experimental import pallas as pl
from jax.experimental.pallas import tpu as pltpu
from jax.experimental.pallas import tpu_sc as plsc

_NB = 4096
_H = 20
_D = 1000
_DP = 1024
_HP = 24
_NW = 32
_BPW = _NB // _NW


@functools.partial(
    pl.kernel,
    mesh=plsc.VectorSubcoreMesh(core_axis_name="c", subcore_axis_name="s"),
    out_type=jax.ShapeDtypeStruct((_NB, _H, _D), jnp.float32),
    scratch_types=[
        pltpu.VMEM((_BPW, _HP), jnp.int32),
        pltpu.VMEM((_HP, 8, 128), jnp.float32),
        pltpu.VMEM((_HP, 8, 128), jnp.float32),
        pltpu.VMEM((_HP, 128), jnp.float32),
        pltpu.VMEM((_HP, 128), jnp.float32),
        pltpu.VMEM((1, _H, _D), jnp.float32),
        pltpu.VMEM((1, _H, _D), jnp.float32),
        pltpu.SemaphoreType.DMA,
        pltpu.SemaphoreType.DMA,
        pltpu.SemaphoreType.DMA,
        pltpu.SemaphoreType.DMA,
        pltpu.SemaphoreType.DMA,
        pltpu.SemaphoreType.DMA,
    ],
    compiler_params=pltpu.CompilerParams(needs_layout_passes=False),
)
def _gather_kernel(idx_hbm, table_hbm, tail_hbm, out_hbm,
                   idx_v, g0, g1, gt0, gt1, p0, p1, gs0, gs1, ts0, ts1, ws0, ws1):
    wid = lax.axis_index("s") * 2 + lax.axis_index("c")
    bbase = wid * _BPW
    pltpu.sync_copy(idx_hbm.at[pl.ds(bbase, _BPW)], idx_v)

    def start_gather(jb, g, gt, sem, tsem):
        pltpu.async_copy(
            table_hbm.at[idx_v.at[jb]], g, sem)
        pltpu.async_copy(
            tail_hbm.at[idx_v.at[jb]], gt, tsem)

    def wait_gather(jb, g, gt, sem, tsem):
        pltpu.make_async_copy(
            table_hbm.at[idx_v.at[jb]], g, sem).wait()
        pltpu.make_async_copy(
            tail_hbm.at[idx_v.at[jb]], gt, tsem).wait()

    def wait_write(p, sem):
        pltpu.make_async_copy(out_hbm.at[pl.ds(0, 1)], p, sem).wait()

    def depad(g, gt, p):
        def row(r, _):
            for s in range(7):
                for k in range(8):
                    p[0, r, pl.ds(s * 128 + k * 16, 16)] = g[r, s, pl.ds(k * 16, 16)]
            for k in range(6):
                p[0, r, pl.ds(896 + k * 16, 16)] = g[r, 7, pl.ds(k * 16, 16)]
            lanes = lax.iota(jnp.int32, 16)
            plsc.store_scatter(
                p, [jnp.zeros((16,), jnp.int32),
                    jnp.full((16,), r, jnp.int32),
                    lanes + (_D - 16)],
                gt[r, pl.ds(112, 16)])
            return 0
        lax.fori_loop(0, _H, row, 0)

    def write_plane(jb, p, sem):
        pltpu.async_copy(p, out_hbm.at[pl.ds(bbase + jb, 1)], sem)

    start_gather(0, g0, gt0, gs0, ts0)

    def body(i, _):
        b0 = 2 * i
        start_gather(b0 + 1, g1, gt1, gs1, ts1)
        wait_gather(b0, g0, gt0, gs0, ts0)

        @pl.when(i > 0)
        def _():
            wait_write(p0, ws0)

        depad(g0, gt0, p0)
        write_plane(b0, p0, ws0)

        @pl.when(i < _BPW // 2 - 1)
        def _():
            start_gather(b0 + 2, g0, gt0, gs0, ts0)

        wait_gather(b0 + 1, g1, gt1, gs1, ts1)

        @pl.when(i > 0)
        def _():
            wait_write(p1, ws1)

        depad(g1, gt1, p1)
        write_plane(b0 + 1, p1, ws1)
        return 0

    lax.fori_loop(0, _BPW // 2, body, 0)
    wait_write(p0, ws0)
    wait_write(p1, ws1)


def kernel(inputs, table):
    idx = jnp.pad(inputs.astype(jnp.int32), ((0, 0), (0, _HP - _H)))
    table_p = jnp.pad(table, ((0, 0), (0, _DP - _D))).reshape(-1, 8, 128)
    tail_t = table[:, _D - 128:]
    return _gather_kernel(idx, table_p, tail_t)


# speedup vs baseline: 2.0485x; 2.0485x over previous
import functools

import jax
import jax.numpy as jnp
from jax import lax
from jax.experimental import pallas as pl
from jax.experimental.pallas import tpu as pltpu
from jax.experimental.pallas import tpu_sc as plsc

_B = 4096 * 20
_D = 1000
_NW = 32
_BPW = _B // _NW
_CH = 64
_NCH = _BPW // _CH


@functools.partial(
    pl.kernel,
    mesh=plsc.VectorSubcoreMesh(core_axis_name="c", subcore_axis_name="s"),
    out_type=jax.ShapeDtypeStruct((_B, _D), jnp.float32),
    scratch_types=[
        pltpu.VMEM((_BPW,), jnp.int32),
        pltpu.VMEM((_CH, _D), jnp.float32),
        pltpu.VMEM((_CH, _D), jnp.float32),
        pltpu.SemaphoreType.DMA,
        pltpu.SemaphoreType.DMA,
    ],
    compiler_params=pltpu.CompilerParams(use_tc_tiling_on_sc=False),
)
def _gather_kernel(idx_hbm, table_hbm, out_hbm, idx_v, buf0, buf1, sem0, sem1):
    wid = lax.axis_index("s") * 2 + lax.axis_index("c")
    base = wid * _BPW
    pltpu.sync_copy(idx_hbm.at[pl.ds(base, _BPW)], idx_v)

    def _start(c, buf, sem):
        pltpu.async_copy(table_hbm.at[idx_v.at[pl.ds(c * _CH, _CH)]], buf, sem)

    def _finish(c, buf, sem):
        pltpu.make_async_copy(table_hbm.at[pl.ds(0, _CH)], buf, sem).wait()
        pltpu.sync_copy(buf, out_hbm.at[pl.ds(base + c * _CH, _CH)])

    _start(0, buf0, sem0)

    def body(i, _):
        c0 = 2 * i
        _start(c0 + 1, buf1, sem1)
        _finish(c0, buf0, sem0)

        @pl.when(i < _NCH // 2 - 1)
        def _():
            _start(c0 + 2, buf0, sem0)

        _finish(c0 + 1, buf1, sem1)
        return 0

    lax.fori_loop(0, _NCH // 2, body, 0)


def kernel(inputs, table):
    idx = inputs.reshape(-1).astype(jnp.int32)
    out = _gather_kernel(idx, table)
    return out.reshape(inputs.shape + (table.shape[1],))
